# baseline (speedup 1.0000x reference)
import functools

import jax
import jax.numpy as jnp
from jax import lax
from jax.experimental import pallas as pl
from jax.experimental.pallas import tpu as pltpu
from jax.experimental.pallas import tpu_sc as plsc

VOCAB = 1000
L = 20
B = 1024
NC, NS = 2, 16
NW = NC * NS
B_PER_W = B // NW


def _sc_gather(table, idx_t):
    mesh = plsc.VectorSubcoreMesh(core_axis_name="c", subcore_axis_name="s")

    @functools.partial(
        pl.kernel,
        mesh=mesh,
        out_type=jax.ShapeDtypeStruct((B, L * VOCAB), jnp.float32),
        scratch_types=[
            pltpu.VMEM((L, B_PER_W), jnp.int32),
            pltpu.VMEM((B_PER_W, VOCAB), jnp.float32),
            pltpu.VMEM((B_PER_W, VOCAB), jnp.float32),
            pltpu.SemaphoreType.DMA,
            pltpu.SemaphoreType.DMA,
            pltpu.SemaphoreType.DMA,
            pltpu.SemaphoreType.DMA,
        ],
        compiler_params=pltpu.CompilerParams(use_tc_tiling_on_sc=False),
    )
    def k(table_hbm, idxt_hbm, out_hbm, idx_v, buf0, buf1, g0, g1, s0, s1):
        wid = lax.axis_index("s") * NC + lax.axis_index("c")
        b0 = wid * B_PER_W
        pltpu.sync_copy(idxt_hbm.at[:, pl.ds(b0, B_PER_W)], idx_v)

        bufs = (buf0, buf1)
        gsems = (g0, g1)
        ssems = (s0, s1)

        def gather(l):
            return pltpu.async_copy(
                table_hbm.at[idx_v.at[l]],
                bufs[l % 2],
                gsems[l % 2],
            )

        def store(l):
            return pltpu.async_copy(
                bufs[l % 2],
                out_hbm.at[pl.ds(b0, B_PER_W), pl.ds(l * VOCAB, VOCAB)],
                ssems[l % 2],
            )

        gd = [None] * L
        sd = [None] * L
        gd[0] = gather(0)
        gd[1] = gather(1)
        gd[0].wait()
        sd[0] = store(0)
        for l in range(1, L):
            sd[l - 1].wait()
            if l + 1 < L:
                gd[l + 1] = gather(l + 1)
            gd[l].wait()
            sd[l] = store(l)
        sd[L - 1].wait()

    return k(table, idx_t)


def kernel(inputs, table):
    idx_t = inputs.astype(jnp.int32).T
    return _sc_gather(table, idx_t)
